# baseline (speedup 1.0000x reference)
import functools

import jax
import jax.numpy as jnp
from jax.experimental import pallas as pl

N = 4096
BM = 256
G = N // BM

F32 = jnp.float32


def _prologue_body(x_ref, w1_ref, b1_ref, w2_ref, b2_ref, w3_ref, b3_ref,
                   wg1_ref, h_ref, g1_ref):
    xb = x_ref[...]
    h1 = jnp.maximum(jnp.dot(xb, w1_ref[...], preferred_element_type=F32)
                     + b1_ref[...], 0.0)
    h1 = jnp.maximum(jnp.dot(h1, w2_ref[...], preferred_element_type=F32)
                     + b2_ref[...], 0.0)
    h = jnp.dot(h1, w3_ref[...], preferred_element_type=F32) + b3_ref[...]
    h_ref[...] = h
    g1_ref[...] = jnp.dot(h, wg1_ref[...], preferred_element_type=F32)


def _pass1_body(hi_ref, ht_ref, g1_ref, t_ref, th_ref, bg1_ref, h2_ref):
    hi = hi_ref[...]
    ht = ht_ref[...]
    t = t_ref[0, 0]
    th = th_ref[0, 0]
    sqi = jnp.sum(hi * hi, axis=1, keepdims=True)
    sqj = jnp.sum(ht * ht, axis=0, keepdims=True)
    hh = jnp.dot(hi, ht, preferred_element_type=F32)
    d2 = jnp.maximum(sqi + sqj - 2.0 * hh, 0.0)
    a = jax.nn.sigmoid(t * (jnp.sqrt(d2) + th))
    agg = jnp.dot(a, g1_ref[...], preferred_element_type=F32) * (1.0 / N)
    h2_ref[...] = jnp.maximum(agg + bg1_ref[...], 0.0)


def _pass2_body(hi_ref, ht_ref, h2_ref, t_ref, th_ref, wg2_ref, bg2_ref,
                wl1_ref, bl1_ref, wl2_ref, bl2_ref, out_ref):
    hi = hi_ref[...]
    ht = ht_ref[...]
    t = t_ref[0, 0]
    th = th_ref[0, 0]
    g2 = jnp.dot(h2_ref[...], wg2_ref[...], preferred_element_type=F32)
    sqi = jnp.sum(hi * hi, axis=1, keepdims=True)
    sqj = jnp.sum(ht * ht, axis=0, keepdims=True)
    hh = jnp.dot(hi, ht, preferred_element_type=F32)
    d2 = jnp.maximum(sqi + sqj - 2.0 * hh, 0.0)
    a = jax.nn.sigmoid(t * (jnp.sqrt(d2) + th))
    agg = jnp.dot(a, g2, preferred_element_type=F32) * (1.0 / N)
    h3 = jnp.maximum(agg + bg2_ref[...], 0.0)
    h4 = jnp.maximum(jnp.dot(h3, wl1_ref[...], preferred_element_type=F32)
                     + bl1_ref[...], 0.0)
    out_ref[...] = (jnp.dot(h4, wl2_ref[...], preferred_element_type=F32)
                    + bl2_ref[...])


def _full(shape):
    nd = len(shape)
    return pl.BlockSpec(shape, lambda i: (0,) * nd)


@functools.partial(jax.jit)
def kernel(x, t, theta, W1, b1, W2, b2, W3, b3, Wg1, bg1, Wg2, bg2,
           Wl1, bl1, Wl2, bl2):
    b1r = b1.reshape(1, -1)
    b2r = b2.reshape(1, -1)
    b3r = b3.reshape(1, -1)
    bg1r = bg1.reshape(1, -1)
    bg2r = bg2.reshape(1, -1)
    bl1r = bl1.reshape(1, -1)
    bl2r = bl2.reshape(1, -1)

    h, g1 = pl.pallas_call(
        _prologue_body,
        grid=(G,),
        in_specs=[
            pl.BlockSpec((BM, 128), lambda i: (i, 0)),
            _full((128, 128)), _full((1, 128)),
            _full((128, 128)), _full((1, 128)),
            _full((128, 32)), _full((1, 32)),
            _full((32, 16)),
        ],
        out_specs=[
            pl.BlockSpec((BM, 32), lambda i: (i, 0)),
            pl.BlockSpec((BM, 16), lambda i: (i, 0)),
        ],
        out_shape=[
            jax.ShapeDtypeStruct((N, 32), F32),
            jax.ShapeDtypeStruct((N, 16), F32),
        ],
    )(x, W1, b1r, W2, b2r, W3, b3r, Wg1)

    ht = h.T

    h2 = pl.pallas_call(
        _pass1_body,
        grid=(G,),
        in_specs=[
            pl.BlockSpec((BM, 32), lambda i: (i, 0)),
            _full((32, N)),
            _full((N, 16)),
            _full((1, 1)), _full((1, 1)),
            _full((1, 16)),
        ],
        out_specs=pl.BlockSpec((BM, 16), lambda i: (i, 0)),
        out_shape=jax.ShapeDtypeStruct((N, 16), F32),
    )(h, ht, g1, t, theta, bg1r)

    out = pl.pallas_call(
        _pass2_body,
        grid=(G,),
        in_specs=[
            pl.BlockSpec((BM, 32), lambda i: (i, 0)),
            _full((32, N)),
            _full((N, 16)),
            _full((1, 1)), _full((1, 1)),
            _full((16, 8)), _full((1, 8)),
            _full((8, 16)), _full((1, 16)),
            _full((16, 2)), _full((1, 2)),
        ],
        out_specs=pl.BlockSpec((BM, 2), lambda i: (i, 0)),
        out_shape=jax.ShapeDtypeStruct((N, 2), F32),
    )(h, ht, h2, t, theta, Wg2, bg2r, Wl1, bl1r, Wl2, bl2r)

    return out


# speedup vs baseline: 1.4782x; 1.4782x over previous
import functools

import jax
import jax.numpy as jnp
from jax.experimental import pallas as pl

N = 4096
BM = 256
G = N // BM

F32 = jnp.float32


def _prologue_body(x_ref, w1_ref, b1_ref, w2_ref, b2_ref, w3_ref, b3_ref,
                   wg1_ref, h_ref, hm2_ref, sq_ref, g1h_ref, csum1_ref):
    xb = x_ref[...]
    h1 = jnp.maximum(jnp.dot(xb, w1_ref[...], preferred_element_type=F32)
                     + b1_ref[...], 0.0)
    h1 = jnp.maximum(jnp.dot(h1, w2_ref[...], preferred_element_type=F32)
                     + b2_ref[...], 0.0)
    h = jnp.dot(h1, w3_ref[...], preferred_element_type=F32) + b3_ref[...]
    h_ref[...] = h
    hm2_ref[...] = h * -2.0
    sq_ref[...] = jnp.sum(h * h, axis=1, keepdims=True)
    g1h = jnp.dot(h, wg1_ref[...], preferred_element_type=F32) * (0.5 / N)
    g1h_ref[...] = g1h
    contrib = jnp.sum(g1h, axis=0, keepdims=True)
    i = pl.program_id(0)

    @pl.when(i == 0)
    def _():
        csum1_ref[...] = contrib

    @pl.when(i > 0)
    def _():
        csum1_ref[...] += contrib


def _pass1_body(hia_ref, htb_ref, g1h_ref, csum1_ref, t_ref, th_ref,
                bg1_ref, wg2_ref, g2h_ref, csum2_ref):
    t = t_ref[0, 0]
    th = th_ref[0, 0]
    p = 0.5 * t
    q = p * th
    d2 = jnp.dot(hia_ref[...], htb_ref[...],
                 preferred_element_type=F32)
    m = jnp.maximum(d2, 1e-30)
    d = m * jax.lax.rsqrt(m)
    tt = jnp.tanh(d * p + q)
    agg = (jnp.dot(tt, g1h_ref[...], preferred_element_type=F32)
           + (csum1_ref[...] + bg1_ref[...]))
    h2 = jnp.maximum(agg, 0.0)
    g2h = jnp.dot(h2, wg2_ref[...], preferred_element_type=F32) * (0.5 / N)
    g2h_ref[...] = g2h
    contrib = jnp.sum(g2h, axis=0, keepdims=True)
    i = pl.program_id(0)

    @pl.when(i == 0)
    def _():
        csum2_ref[...] = contrib

    @pl.when(i > 0)
    def _():
        csum2_ref[...] += contrib


def _pass2_body(hia_ref, htb_ref, g2h_ref, csum2_ref, t_ref, th_ref,
                bg2_ref, wl1_ref, bl1_ref, wl2_ref, bl2_ref, out_ref):
    t = t_ref[0, 0]
    th = th_ref[0, 0]
    p = 0.5 * t
    q = p * th
    d2 = jnp.dot(hia_ref[...], htb_ref[...],
                 preferred_element_type=F32)
    m = jnp.maximum(d2, 1e-30)
    d = m * jax.lax.rsqrt(m)
    tt = jnp.tanh(d * p + q)
    agg = (jnp.dot(tt, g2h_ref[...], preferred_element_type=F32)
           + (csum2_ref[...] + bg2_ref[...]))
    h3 = jnp.maximum(agg, 0.0)
    h4 = jnp.maximum(jnp.dot(h3, wl1_ref[...], preferred_element_type=F32)
                     + bl1_ref[...], 0.0)
    out_ref[...] = (jnp.dot(h4, wl2_ref[...], preferred_element_type=F32)
                    + bl2_ref[...])


def _full(shape):
    nd = len(shape)
    return pl.BlockSpec(shape, lambda i: (0,) * nd)


@functools.partial(jax.jit)
def kernel(x, t, theta, W1, b1, W2, b2, W3, b3, Wg1, bg1, Wg2, bg2,
           Wl1, bl1, Wl2, bl2):
    b1r = b1.reshape(1, -1)
    b2r = b2.reshape(1, -1)
    b3r = b3.reshape(1, -1)
    bg1r = bg1.reshape(1, -1)
    bg2r = bg2.reshape(1, -1)
    bl1r = bl1.reshape(1, -1)
    bl2r = bl2.reshape(1, -1)

    h, hm2, sq, g1h, csum1 = pl.pallas_call(
        _prologue_body,
        grid=(G,),
        in_specs=[
            pl.BlockSpec((BM, 128), lambda i: (i, 0)),
            _full((128, 128)), _full((1, 128)),
            _full((128, 128)), _full((1, 128)),
            _full((128, 32)), _full((1, 32)),
            _full((32, 16)),
        ],
        out_specs=[
            pl.BlockSpec((BM, 32), lambda i: (i, 0)),
            pl.BlockSpec((BM, 32), lambda i: (i, 0)),
            pl.BlockSpec((BM, 1), lambda i: (i, 0)),
            pl.BlockSpec((BM, 16), lambda i: (i, 0)),
            pl.BlockSpec((1, 16), lambda i: (0, 0)),
        ],
        out_shape=[
            jax.ShapeDtypeStruct((N, 32), F32),
            jax.ShapeDtypeStruct((N, 32), F32),
            jax.ShapeDtypeStruct((N, 1), F32),
            jax.ShapeDtypeStruct((N, 16), F32),
            jax.ShapeDtypeStruct((1, 16), F32),
        ],
    )(x, W1, b1r, W2, b2r, W3, b3r, Wg1)

    ones = jnp.ones((N, 1), F32)
    hia = jnp.concatenate([hm2, sq, ones], axis=1)
    htb = jnp.concatenate([h, ones, sq], axis=1).T

    g2h, csum2 = pl.pallas_call(
        _pass1_body,
        grid=(G,),
        in_specs=[
            pl.BlockSpec((BM, 34), lambda i: (i, 0)),
            _full((34, N)),
            _full((N, 16)),
            _full((1, 16)),
            _full((1, 1)), _full((1, 1)),
            _full((1, 16)),
            _full((16, 8)),
        ],
        out_specs=[
            pl.BlockSpec((BM, 8), lambda i: (i, 0)),
            pl.BlockSpec((1, 8), lambda i: (0, 0)),
        ],
        out_shape=[
            jax.ShapeDtypeStruct((N, 8), F32),
            jax.ShapeDtypeStruct((1, 8), F32),
        ],
    )(hia, htb, g1h, csum1, t, theta, bg1r, Wg2)

    out = pl.pallas_call(
        _pass2_body,
        grid=(G,),
        in_specs=[
            pl.BlockSpec((BM, 34), lambda i: (i, 0)),
            _full((34, N)),
            _full((N, 8)),
            _full((1, 8)),
            _full((1, 1)), _full((1, 1)),
            _full((1, 8)),
            _full((8, 16)), _full((1, 16)),
            _full((16, 2)), _full((1, 2)),
        ],
        out_specs=pl.BlockSpec((BM, 2), lambda i: (i, 0)),
        out_shape=jax.ShapeDtypeStruct((N, 2), F32),
    )(hia, htb, g2h, csum2, t, theta, bg2r, Wl1, bl1r, Wl2, bl2r)

    return out


# speedup vs baseline: 1.6879x; 1.1418x over previous
import functools

import jax
import jax.numpy as jnp
from jax.experimental import pallas as pl
from jax.experimental.pallas import tpu as pltpu

N = 4096
BM = 256
G = N // BM
KA = 34

F32 = jnp.float32


def _body(x_ref, w1_ref, b1_ref, w2_ref, b2_ref, w3_ref, b3_ref, wg1_ref,
          t_ref, th_ref, bg1_ref, wg2_ref, bg2_ref, wl1_ref, bl1_ref,
          wl2_ref, bl2_ref, out_ref,
          hia_s, htb_s, g1h_s, g2h_s, cs1_s, cs2_s):
    i = pl.program_id(0)
    t = t_ref[0, 0]
    th = th_ref[0, 0]
    p = 0.5 * t
    q = p * th

    @pl.when(i < G)
    def _prologue():
        b = i
        xb = x_ref[...]
        h1 = jnp.maximum(jnp.dot(xb, w1_ref[...], preferred_element_type=F32)
                         + b1_ref[...], 0.0)
        h1 = jnp.maximum(jnp.dot(h1, w2_ref[...], preferred_element_type=F32)
                         + b2_ref[...], 0.0)
        h = (jnp.dot(h1, w3_ref[...], preferred_element_type=F32)
             + b3_ref[...])
        sq = jnp.sum(h * h, axis=1, keepdims=True)
        ones = jnp.ones((BM, 1), F32)
        hia = jnp.concatenate([h * -2.0, sq, ones], axis=1)
        hib = jnp.concatenate([h, ones, sq], axis=1)
        hia_s[pl.ds(b * BM, BM), :] = hia
        htb_s[:, pl.ds(b * BM, BM)] = hib.T
        g1h = (jnp.dot(h, wg1_ref[...], preferred_element_type=F32)
               * (0.5 / N))
        g1h_s[pl.ds(b * BM, BM), :] = g1h
        contrib = jnp.sum(g1h, axis=0, keepdims=True)

        @pl.when(b == 0)
        def _():
            cs1_s[...] = contrib

        @pl.when(b > 0)
        def _():
            cs1_s[...] += contrib

    @pl.when((i >= G) & (i < 2 * G))
    def _pass1():
        b = i - G
        hia = hia_s[pl.ds(b * BM, BM), :]
        d2 = jnp.dot(hia, htb_s[...], preferred_element_type=F32)
        m = jnp.maximum(d2, 1e-30)
        d = m * jax.lax.rsqrt(m)
        tt = jnp.tanh(d * p + q)
        agg = (jnp.dot(tt, g1h_s[...], preferred_element_type=F32)
               + (cs1_s[...] + bg1_ref[...]))
        h2 = jnp.maximum(agg, 0.0)
        g2h = (jnp.dot(h2, wg2_ref[...], preferred_element_type=F32)
               * (0.5 / N))
        g2h_s[pl.ds(b * BM, BM), :] = g2h
        contrib = jnp.sum(g2h, axis=0, keepdims=True)

        @pl.when(b == 0)
        def _():
            cs2_s[...] = contrib

        @pl.when(b > 0)
        def _():
            cs2_s[...] += contrib

    @pl.when(i >= 2 * G)
    def _pass2():
        b = i - 2 * G
        hia = hia_s[pl.ds(b * BM, BM), :]
        d2 = jnp.dot(hia, htb_s[...], preferred_element_type=F32)
        m = jnp.maximum(d2, 1e-30)
        d = m * jax.lax.rsqrt(m)
        tt = jnp.tanh(d * p + q)
        agg = (jnp.dot(tt, g2h_s[...], preferred_element_type=F32)
               + (cs2_s[...] + bg2_ref[...]))
        h3 = jnp.maximum(agg, 0.0)
        h4 = jnp.maximum(jnp.dot(h3, wl1_ref[...], preferred_element_type=F32)
                         + bl1_ref[...], 0.0)
        out_ref[...] = (jnp.dot(h4, wl2_ref[...], preferred_element_type=F32)
                        + bl2_ref[...])


def _full(shape):
    nd = len(shape)
    return pl.BlockSpec(shape, lambda i: (0,) * nd)


@functools.partial(jax.jit)
def kernel(x, t, theta, W1, b1, W2, b2, W3, b3, Wg1, bg1, Wg2, bg2,
           Wl1, bl1, Wl2, bl2):
    b1r = b1.reshape(1, -1)
    b2r = b2.reshape(1, -1)
    b3r = b3.reshape(1, -1)
    bg1r = bg1.reshape(1, -1)
    bg2r = bg2.reshape(1, -1)
    bl1r = bl1.reshape(1, -1)
    bl2r = bl2.reshape(1, -1)

    out = pl.pallas_call(
        _body,
        grid=(3 * G,),
        in_specs=[
            pl.BlockSpec((BM, 128), lambda i: (jnp.minimum(i, G - 1), 0)),
            _full((128, 128)), _full((1, 128)),
            _full((128, 128)), _full((1, 128)),
            _full((128, 32)), _full((1, 32)),
            _full((32, 16)),
            _full((1, 1)), _full((1, 1)),
            _full((1, 16)),
            _full((16, 8)), _full((1, 8)),
            _full((8, 16)), _full((1, 16)),
            _full((16, 2)), _full((1, 2)),
        ],
        out_specs=pl.BlockSpec((BM, 2),
                               lambda i: (jnp.maximum(i - 2 * G, 0), 0)),
        out_shape=jax.ShapeDtypeStruct((N, 2), F32),
        scratch_shapes=[
            pltpu.VMEM((N, KA), F32),
            pltpu.VMEM((KA, N), F32),
            pltpu.VMEM((N, 16), F32),
            pltpu.VMEM((N, 8), F32),
            pltpu.VMEM((1, 16), F32),
            pltpu.VMEM((1, 8), F32),
        ],
    )(x, W1, b1r, W2, b2r, W3, b3r, Wg1, t, theta, bg1r, Wg2, bg2r,
      Wl1, bl1r, Wl2, bl2r)

    return out


# speedup vs baseline: 2.0940x; 1.2406x over previous
import functools

import jax
import jax.numpy as jnp
from jax.experimental import pallas as pl
from jax.experimental.pallas import tpu as pltpu

N = 4096
BM = 512
G = N // BM
KA = 34

F32 = jnp.float32


def _body(x_ref, w1_ref, b1_ref, w2_ref, b2_ref, w3_ref, b3_ref, wg1_ref,
          t_ref, th_ref, bg1_ref, wg2_ref, bg2_ref, wl1_ref, bl1_ref,
          wl2_ref, bl2_ref, out_ref,
          hia_s, htb_s, g1h_s, g2h_s, cs1_s, cs2_s):
    i = pl.program_id(0)
    t = t_ref[0, 0]
    th = th_ref[0, 0]
    p = 0.5 * t
    q = p * th

    @pl.when(i < G)
    def _prologue():
        b = i
        xb = x_ref[...]
        h1 = jnp.maximum(jnp.dot(xb, w1_ref[...], preferred_element_type=F32)
                         + b1_ref[...], 0.0)
        h1 = jnp.maximum(jnp.dot(h1, w2_ref[...], preferred_element_type=F32)
                         + b2_ref[...], 0.0)
        h = (jnp.dot(h1, w3_ref[...], preferred_element_type=F32)
             + b3_ref[...])
        sq = jnp.sum(h * h, axis=1, keepdims=True)
        ones = jnp.ones((BM, 1), F32)
        hia = jnp.concatenate([h * -2.0, sq, ones], axis=1)
        hib = jnp.concatenate([h, ones, sq], axis=1)
        hia_s[pl.ds(b * BM, BM), :] = hia
        htb_s[:, pl.ds(b * BM, BM)] = hib.T
        g1h = (jnp.dot(h, wg1_ref[...], preferred_element_type=F32)
               * (0.5 / N))
        g1h_s[pl.ds(b * BM, BM), :] = g1h
        contrib = jnp.sum(g1h, axis=0, keepdims=True)

        @pl.when(b == 0)
        def _():
            cs1_s[...] = contrib

        @pl.when(b > 0)
        def _():
            cs1_s[...] += contrib

    @pl.when((i >= G) & (i < 2 * G))
    def _pass1():
        b = i - G
        hia = hia_s[pl.ds(b * BM, BM), :]
        d2 = jnp.dot(hia, htb_s[...], preferred_element_type=F32)
        m = jnp.maximum(d2, 1e-30)
        d = m * jax.lax.rsqrt(m)
        tt = jnp.tanh(d * p + q)
        agg = (jnp.dot(tt, g1h_s[...], preferred_element_type=F32)
               + (cs1_s[...] + bg1_ref[...]))
        h2 = jnp.maximum(agg, 0.0)
        g2h = (jnp.dot(h2, wg2_ref[...], preferred_element_type=F32)
               * (0.5 / N))
        g2h_s[pl.ds(b * BM, BM), :] = g2h
        contrib = jnp.sum(g2h, axis=0, keepdims=True)

        @pl.when(b == 0)
        def _():
            cs2_s[...] = contrib

        @pl.when(b > 0)
        def _():
            cs2_s[...] += contrib

    @pl.when(i >= 2 * G)
    def _pass2():
        b = i - 2 * G
        hia = hia_s[pl.ds(b * BM, BM), :]
        d2 = jnp.dot(hia, htb_s[...], preferred_element_type=F32)
        m = jnp.maximum(d2, 1e-30)
        d = m * jax.lax.rsqrt(m)
        tt = jnp.tanh(d * p + q)
        agg = (jnp.dot(tt, g2h_s[...], preferred_element_type=F32)
               + (cs2_s[...] + bg2_ref[...]))
        h3 = jnp.maximum(agg, 0.0)
        h4 = jnp.maximum(jnp.dot(h3, wl1_ref[...], preferred_element_type=F32)
                         + bl1_ref[...], 0.0)
        out_ref[...] = (jnp.dot(h4, wl2_ref[...], preferred_element_type=F32)
                        + bl2_ref[...])


def _full(shape):
    nd = len(shape)
    return pl.BlockSpec(shape, lambda i: (0,) * nd)


@functools.partial(jax.jit)
def kernel(x, t, theta, W1, b1, W2, b2, W3, b3, Wg1, bg1, Wg2, bg2,
           Wl1, bl1, Wl2, bl2):
    b1r = b1.reshape(1, -1)
    b2r = b2.reshape(1, -1)
    b3r = b3.reshape(1, -1)
    bg1r = bg1.reshape(1, -1)
    bg2r = bg2.reshape(1, -1)
    bl1r = bl1.reshape(1, -1)
    bl2r = bl2.reshape(1, -1)

    out = pl.pallas_call(
        _body,
        grid=(3 * G,),
        in_specs=[
            pl.BlockSpec((BM, 128), lambda i: (jnp.minimum(i, G - 1), 0)),
            _full((128, 128)), _full((1, 128)),
            _full((128, 128)), _full((1, 128)),
            _full((128, 32)), _full((1, 32)),
            _full((32, 16)),
            _full((1, 1)), _full((1, 1)),
            _full((1, 16)),
            _full((16, 8)), _full((1, 8)),
            _full((8, 16)), _full((1, 16)),
            _full((16, 2)), _full((1, 2)),
        ],
        out_specs=pl.BlockSpec((BM, 2),
                               lambda i: (jnp.maximum(i - 2 * G, 0), 0)),
        out_shape=jax.ShapeDtypeStruct((N, 2), F32),
        scratch_shapes=[
            pltpu.VMEM((N, KA), F32),
            pltpu.VMEM((KA, N), F32),
            pltpu.VMEM((N, 16), F32),
            pltpu.VMEM((N, 8), F32),
            pltpu.VMEM((1, 16), F32),
            pltpu.VMEM((1, 8), F32),
        ],
    )(x, W1, b1r, W2, b2r, W3, b3r, Wg1, t, theta, bg1r, Wg2, bg2r,
      Wl1, bl1r, Wl2, bl2r)

    return out


# speedup vs baseline: 2.3894x; 1.1411x over previous
import functools

import jax
import jax.numpy as jnp
from jax.experimental import pallas as pl
from jax.experimental.pallas import tpu as pltpu

N = 4096
BM = 1024
G = N // BM
KA = 34

F32 = jnp.float32


def _body(x_ref, w1_ref, b1_ref, w2_ref, b2_ref, w3_ref, b3_ref, wg1_ref,
          t_ref, th_ref, bg1_ref, wg2_ref, bg2_ref, wl1_ref, bl1_ref,
          wl2_ref, bl2_ref, out_ref,
          hia_s, htb_s, g1h_s, g2h_s, cs1_s, cs2_s):
    i = pl.program_id(0)
    t = t_ref[0, 0]
    th = th_ref[0, 0]
    p = 0.5 * t
    q = p * th

    @pl.when(i < G)
    def _prologue():
        b = i
        xb = x_ref[...]
        h1 = jnp.maximum(jnp.dot(xb, w1_ref[...], preferred_element_type=F32)
                         + b1_ref[...], 0.0)
        h1 = jnp.maximum(jnp.dot(h1, w2_ref[...], preferred_element_type=F32)
                         + b2_ref[...], 0.0)
        h = (jnp.dot(h1, w3_ref[...], preferred_element_type=F32)
             + b3_ref[...])
        sq = jnp.sum(h * h, axis=1, keepdims=True)
        ones = jnp.ones((BM, 1), F32)
        hia = jnp.concatenate([h * -2.0, sq, ones], axis=1)
        hib = jnp.concatenate([h, ones, sq], axis=1)
        hia_s[pl.ds(b * BM, BM), :] = hia
        htb_s[:, pl.ds(b * BM, BM)] = hib.T
        g1h = (jnp.dot(h, wg1_ref[...], preferred_element_type=F32)
               * (0.5 / N))
        g1h_s[pl.ds(b * BM, BM), :] = g1h
        contrib = jnp.sum(g1h, axis=0, keepdims=True)

        @pl.when(b == 0)
        def _():
            cs1_s[...] = contrib

        @pl.when(b > 0)
        def _():
            cs1_s[...] += contrib

    @pl.when((i >= G) & (i < 2 * G))
    def _pass1():
        b = i - G
        hia = hia_s[pl.ds(b * BM, BM), :]
        d2 = jnp.dot(hia, htb_s[...], preferred_element_type=F32)
        m = jnp.maximum(d2, 1e-30)
        d = m * jax.lax.rsqrt(m)
        tt = jnp.tanh(d * p + q)
        agg = (jnp.dot(tt, g1h_s[...], preferred_element_type=F32)
               + (cs1_s[...] + bg1_ref[...]))
        h2 = jnp.maximum(agg, 0.0)
        g2h = (jnp.dot(h2, wg2_ref[...], preferred_element_type=F32)
               * (0.5 / N))
        g2h_s[pl.ds(b * BM, BM), :] = g2h
        contrib = jnp.sum(g2h, axis=0, keepdims=True)

        @pl.when(b == 0)
        def _():
            cs2_s[...] = contrib

        @pl.when(b > 0)
        def _():
            cs2_s[...] += contrib

    @pl.when(i >= 2 * G)
    def _pass2():
        b = i - 2 * G
        hia = hia_s[pl.ds(b * BM, BM), :]
        d2 = jnp.dot(hia, htb_s[...], preferred_element_type=F32)
        m = jnp.maximum(d2, 1e-30)
        d = m * jax.lax.rsqrt(m)
        tt = jnp.tanh(d * p + q)
        agg = (jnp.dot(tt, g2h_s[...], preferred_element_type=F32)
               + (cs2_s[...] + bg2_ref[...]))
        h3 = jnp.maximum(agg, 0.0)
        h4 = jnp.maximum(jnp.dot(h3, wl1_ref[...], preferred_element_type=F32)
                         + bl1_ref[...], 0.0)
        out_ref[...] = (jnp.dot(h4, wl2_ref[...], preferred_element_type=F32)
                        + bl2_ref[...])


def _full(shape):
    nd = len(shape)
    return pl.BlockSpec(shape, lambda i: (0,) * nd)


@functools.partial(jax.jit)
def kernel(x, t, theta, W1, b1, W2, b2, W3, b3, Wg1, bg1, Wg2, bg2,
           Wl1, bl1, Wl2, bl2):
    b1r = b1.reshape(1, -1)
    b2r = b2.reshape(1, -1)
    b3r = b3.reshape(1, -1)
    bg1r = bg1.reshape(1, -1)
    bg2r = bg2.reshape(1, -1)
    bl1r = bl1.reshape(1, -1)
    bl2r = bl2.reshape(1, -1)

    out = pl.pallas_call(
        _body,
        grid=(3 * G,),
        in_specs=[
            pl.BlockSpec((BM, 128), lambda i: (jnp.minimum(i, G - 1), 0)),
            _full((128, 128)), _full((1, 128)),
            _full((128, 128)), _full((1, 128)),
            _full((128, 32)), _full((1, 32)),
            _full((32, 16)),
            _full((1, 1)), _full((1, 1)),
            _full((1, 16)),
            _full((16, 8)), _full((1, 8)),
            _full((8, 16)), _full((1, 16)),
            _full((16, 2)), _full((1, 2)),
        ],
        out_specs=pl.BlockSpec((BM, 2),
                               lambda i: (jnp.maximum(i - 2 * G, 0), 0)),
        out_shape=jax.ShapeDtypeStruct((N, 2), F32),
        scratch_shapes=[
            pltpu.VMEM((N, KA), F32),
            pltpu.VMEM((KA, N), F32),
            pltpu.VMEM((N, 16), F32),
            pltpu.VMEM((N, 8), F32),
            pltpu.VMEM((1, 16), F32),
            pltpu.VMEM((1, 8), F32),
        ],
    )(x, W1, b1r, W2, b2r, W3, b3r, Wg1, t, theta, bg1r, Wg2, bg2r,
      Wl1, bl1r, Wl2, bl2r)

    return out
